# baseline (speedup 1.0000x reference)
import functools

import jax
import jax.numpy as jnp
from jax import lax
from jax.experimental import pallas as pl
from jax.experimental.pallas import tpu as pltpu
from jax.experimental.pallas import tpu_sc as plsc

_B, _S, _H = 4, 4096, 2048
_NC, _NS = 2, 16
_NW = _NC * _NS
_ROWS = _B * _S
_RPW = _ROWS // _NW
_CHUNK = 16


def _sc_body(x_hbm, pe_hbm, out_hbm, xbuf, pebuf):
    c = lax.axis_index("c")
    s = lax.axis_index("s")
    wid = s * _NC + c
    base = wid * _RPW
    pe_base = lax.rem(base, _S)

    def chunk(ci, carry):
        r0 = base + ci * _CHUNK
        p0 = pe_base + ci * _CHUNK
        pltpu.sync_copy(x_hbm.at[pl.ds(r0, _CHUNK)], xbuf)
        pltpu.sync_copy(pe_hbm.at[pl.ds(p0, _CHUNK)], pebuf)

        def row(r, rcarry):
            for j in range(_H // 16):
                sl = pl.ds(j * 16, 16)
                xbuf[r, sl] = xbuf[r, sl] + pebuf[r, sl]
            return rcarry

        lax.fori_loop(0, _CHUNK, row, 0)
        pltpu.sync_copy(xbuf, out_hbm.at[pl.ds(r0, _CHUNK)])
        return carry

    lax.fori_loop(0, _RPW // _CHUNK, chunk, 0)


def kernel(x, pe):
    xf = x.reshape(_ROWS, _H)
    mesh = plsc.VectorSubcoreMesh(core_axis_name="c", subcore_axis_name="s")
    out = pl.kernel(
        _sc_body,
        out_type=jax.ShapeDtypeStruct((_ROWS, _H), jnp.float32),
        mesh=mesh,
        scratch_types=[
            pltpu.VMEM((_CHUNK, _H), jnp.float32),
            pltpu.VMEM((_CHUNK, _H), jnp.float32),
        ],
    )(xf, pe)
    return out.reshape(_B, _S, _H)


# speedup vs baseline: 1.5820x; 1.5820x over previous
import functools

import jax
import jax.numpy as jnp
from jax import lax
from jax.experimental import pallas as pl
from jax.experimental.pallas import tpu as pltpu
from jax.experimental.pallas import tpu_sc as plsc

_B, _S, _H = 4, 4096, 2048
_NC, _NS = 2, 16
_NW = _NC * _NS
_SC_B = 1
_TC_B = _B - _SC_B
_SC_ROWS = _SC_B * _S
_RPW = _SC_ROWS // _NW
_CHUNK = 16
_BS = 1024


def _tc_body(x_ref, pe_ref, o_ref):
    o_ref[...] = x_ref[...] + pe_ref[...]


def _sc_body(x_hbm, pe_hbm, out_hbm, xbuf, pebuf):
    c = lax.axis_index("c")
    s = lax.axis_index("s")
    wid = s * _NC + c
    base = wid * _RPW
    pe_base = lax.rem(_TC_B * _S + base, _S)

    def chunk(ci, carry):
        r0 = base + ci * _CHUNK
        p0 = pe_base + ci * _CHUNK
        pltpu.sync_copy(x_hbm.at[pl.ds(_TC_B * _S + r0, _CHUNK)], xbuf)
        pltpu.sync_copy(pe_hbm.at[pl.ds(p0, _CHUNK)], pebuf)

        def row(r, rcarry):
            for j in range(_H // 16):
                sl = pl.ds(j * 16, 16)
                xbuf[r, sl] = xbuf[r, sl] + pebuf[r, sl]
            return rcarry

        lax.fori_loop(0, _CHUNK, row, 0)
        pltpu.sync_copy(xbuf, out_hbm.at[pl.ds(r0, _CHUNK)])
        return carry

    lax.fori_loop(0, _RPW // _CHUNK, chunk, 0)


def kernel(x, pe):
    B, S, H = x.shape
    tc_out = pl.pallas_call(
        _tc_body,
        grid=(S // _BS, _TC_B),
        in_specs=[
            pl.BlockSpec((1, _BS, H), lambda s, b: (b, s, 0)),
            pl.BlockSpec((_BS, H), lambda s, b: (s, 0)),
        ],
        out_specs=pl.BlockSpec((1, _BS, H), lambda s, b: (b, s, 0)),
        out_shape=jax.ShapeDtypeStruct((_TC_B, S, H), x.dtype),
    )(x, pe)

    mesh = plsc.VectorSubcoreMesh(core_axis_name="c", subcore_axis_name="s")
    sc_out = pl.kernel(
        _sc_body,
        out_type=jax.ShapeDtypeStruct((_SC_ROWS, _H), jnp.float32),
        mesh=mesh,
        scratch_types=[
            pltpu.VMEM((_CHUNK, _H), jnp.float32),
            pltpu.VMEM((_CHUNK, _H), jnp.float32),
        ],
    )(x.reshape(_B * _S, _H), pe)

    return jnp.concatenate([tc_out, sc_out.reshape(_SC_B, S, H)], axis=0)


# speedup vs baseline: 3.5932x; 2.2714x over previous
import jax
import jax.numpy as jnp
from jax.experimental import pallas as pl
from jax.experimental.pallas import tpu as pltpu
from jax import lax

_B, _S, _H = 4, 4096, 2048
_ROWS = _B * _S
_R = 256
_NCH = _ROWS // _R
_NPE = _S // _R
_NBUF = 4


def _body(x_hbm, pe_hbm, o_hbm, pev, xb, ob, sx, sp, so):
    for i in range(_NBUF):
        pltpu.make_async_copy(
            pe_hbm.at[pl.ds(i * _R, _R)], pev.at[pl.ds(i * _R, _R)], sp.at[i]
        ).start()
        pltpu.make_async_copy(
            x_hbm.at[pl.ds(i * _R, _R)], xb.at[i], sx.at[i]
        ).start()

    def step(c, carry):
        s = lax.rem(c, _NBUF)
        p = lax.rem(c, _NPE)

        @pl.when(c + _NBUF < _NPE)
        def _():
            i = c + _NBUF
            pltpu.make_async_copy(
                pe_hbm.at[pl.ds(i * _R, _R)], pev.at[pl.ds(i * _R, _R)], sp.at[i]
            ).start()

        pltpu.make_async_copy(
            x_hbm.at[pl.ds(c * _R, _R)], xb.at[s], sx.at[s]
        ).wait()

        @pl.when(c < _NPE)
        def _():
            pltpu.make_async_copy(
                pe_hbm.at[pl.ds(p * _R, _R)], pev.at[pl.ds(p * _R, _R)], sp.at[p]
            ).wait()

        @pl.when(c >= _NBUF)
        def _():
            pltpu.make_async_copy(
                ob.at[s], o_hbm.at[pl.ds((c - _NBUF) * _R, _R)], so.at[s]
            ).wait()

        ob[s] = xb[s][...] + pev[pl.ds(p * _R, _R), :]

        pltpu.make_async_copy(
            ob.at[s], o_hbm.at[pl.ds(c * _R, _R)], so.at[s]
        ).start()

        @pl.when(c + _NBUF < _NCH)
        def _():
            i = c + _NBUF
            pltpu.make_async_copy(
                x_hbm.at[pl.ds(i * _R, _R)], xb.at[lax.rem(i, _NBUF)],
                sx.at[lax.rem(i, _NBUF)],
            ).start()

        return carry

    lax.fori_loop(0, _NCH, step, 0)

    for i in range(_NBUF):
        c = _NCH - _NBUF + i
        pltpu.make_async_copy(
            ob.at[c % _NBUF], o_hbm.at[pl.ds(c * _R, _R)], so.at[c % _NBUF]
        ).wait()


def kernel(x, pe):
    out = pl.pallas_call(
        _body,
        in_specs=[
            pl.BlockSpec(memory_space=pl.ANY),
            pl.BlockSpec(memory_space=pl.ANY),
        ],
        out_specs=pl.BlockSpec(memory_space=pl.ANY),
        out_shape=jax.ShapeDtypeStruct((_ROWS, _H), x.dtype),
        scratch_shapes=[
            pltpu.VMEM((_S, _H), jnp.float32),
            pltpu.VMEM((_NBUF, _R, _H), jnp.float32),
            pltpu.VMEM((_NBUF, _R, _H), jnp.float32),
            pltpu.SemaphoreType.DMA((_NBUF,)),
            pltpu.SemaphoreType.DMA((_NPE,)),
            pltpu.SemaphoreType.DMA((_NBUF,)),
        ],
    )(x.reshape(_ROWS, _H), pe)
    return out.reshape(_B, _S, _H)


# speedup vs baseline: 3.5992x; 1.0017x over previous
import jax
import jax.numpy as jnp
from jax.experimental import pallas as pl


_BS = 1024


def _add_kernel(x_ref, pe_ref, o_ref):
    o_ref[...] = x_ref[...] + pe_ref[...]


def kernel(x, pe):
    B, S, H = x.shape
    grid = (S // _BS, B)
    return pl.pallas_call(
        _add_kernel,
        grid=grid,
        in_specs=[
            pl.BlockSpec((1, _BS, H), lambda s, b: (b, s, 0)),
            pl.BlockSpec((_BS, H), lambda s, b: (s, 0)),
        ],
        out_specs=pl.BlockSpec((1, _BS, H), lambda s, b: (b, s, 0)),
        out_shape=jax.ShapeDtypeStruct((B, S, H), x.dtype),
    )(x, pe)
